# Initial kernel scaffold; baseline (speedup 1.0000x reference)
import jax
import jax.numpy as jnp
from jax.experimental import pallas as pl


def kernel(idx, targets, table):
    raise NotImplementedError("write your pallas kernel here")


# speedup vs baseline: 1.6127x; 1.6127x over previous
import functools

import jax
import jax.numpy as jnp
from jax import lax
from jax.experimental import pallas as pl
from jax.experimental.pallas import tpu as pltpu
from jax.experimental.pallas import tpu_sc as plsc


def _lse_body(table_ref, lse_ref):
    t = table_ref[...]
    m = jnp.max(t, axis=1)
    s = jnp.sum(jnp.exp(t - m[:, None]), axis=1)
    lse_ref[...] = m + jnp.log(s)


def _compute_lse(table):
    V = table.shape[0]
    return pl.pallas_call(
        _lse_body,
        out_shape=jax.ShapeDtypeStruct((V,), jnp.float32),
    )(table)


@functools.lru_cache(maxsize=None)
def _make_sc_gather(N, V, C):
    NC, NS = 2, 16
    NW = NC * NS
    assert N % NW == 0
    BPW = N // NW
    CH = 32
    assert BPW % CH == 0 and CH % 16 == 0
    NCHUNK = BPW // CH
    G = CH // 16

    mesh = plsc.VectorSubcoreMesh(core_axis_name="c", subcore_axis_name="s")

    @functools.partial(
        pl.kernel,
        mesh=mesh,
        compiler_params=pltpu.CompilerParams(
            needs_layout_passes=False, use_tc_tiling_on_sc=False
        ),
        out_type=(
            jax.ShapeDtypeStruct((N, C), jnp.float32),
            jax.ShapeDtypeStruct((NW, 16), jnp.float32),
        ),
        scratch_types=[
            pltpu.VMEM((BPW,), jnp.int32),
            pltpu.VMEM((BPW,), jnp.int32),
            pltpu.VMEM((V,), jnp.float32),
            pltpu.VMEM((CH, C), jnp.float32),
            pltpu.VMEM((16,), jnp.float32),
            pltpu.SemaphoreType.DMA,
        ],
    )
    def sc_kernel(table_hbm, idx_hbm, tgt_hbm, lse_hbm, out_hbm, part_hbm,
                  idx_v, tgt_v, lse_v, rows_v, acc_v, sem):
        wid = lax.axis_index("s") * NC + lax.axis_index("c")
        base = wid * BPW
        pltpu.sync_copy(idx_hbm.at[pl.ds(base, BPW)], idx_v)
        pltpu.sync_copy(tgt_hbm.at[pl.ds(base, BPW)], tgt_v)
        pltpu.sync_copy(lse_hbm, lse_v)
        acc_v[...] = jnp.zeros((16,), jnp.float32)

        @pl.loop(0, NCHUNK)
        def _chunk(ci):
            off = ci * CH
            pltpu.async_copy(
                table_hbm.at[idx_v.at[pl.ds(off, CH)]], rows_v, sem
            ).wait()
            for g in range(G):
                j0 = off + g * 16
                ivec = idx_v[pl.ds(j0, 16)]
                tvec = tgt_v[pl.ds(j0, 16)]
                lsev = plsc.load_gather(lse_v, [ivec])
                lrow = lax.broadcasted_iota(jnp.int32, (16,), 0) + g * 16
                tval = plsc.load_gather(rows_v, [lrow, tvec])
                acc_v[...] = acc_v[...] + (lsev - tval)
            pltpu.sync_copy(rows_v, out_hbm.at[pl.ds(base + off, CH)])

        pltpu.sync_copy(acc_v, part_hbm.at[wid])

    return sc_kernel


def _make_fin(N):
    def _fin_body(part_ref, loss_ref):
        loss_ref[...] = (jnp.sum(part_ref[...]) / N).reshape(1, 1)

    return pl.pallas_call(
        _fin_body,
        out_shape=jax.ShapeDtypeStruct((1, 1), jnp.float32),
    )


def kernel(idx, targets, table):
    B, T = idx.shape
    V, C = table.shape
    N = B * T
    idx_flat = idx.reshape(N).astype(jnp.int32)
    tgt_flat = targets.reshape(N).astype(jnp.int32)
    table = table.astype(jnp.float32)

    lse = _compute_lse(table)
    logits, partials = _make_sc_gather(N, V, C)(table, idx_flat, tgt_flat, lse)
    loss = _make_fin(N)(partials).reshape(())
    return (logits, loss)
